# Initial kernel scaffold; baseline (speedup 1.0000x reference)
import jax
import jax.numpy as jnp
from jax.experimental import pallas as pl


def kernel(x, a2a, wk, wqv_w, wqv_b, fanin_w, fanin_b, layer, pas):
    raise NotImplementedError("write your pallas kernel here")


# speedup vs baseline: 8.8381x; 8.8381x over previous
import jax
import jax.numpy as jnp
from jax.experimental import pallas as pl
from jax.experimental.pallas import tpu as pltpu

P = 512
NV = 500
NH = 8
DM = 768
SCALE = 1.0 / (DM ** 0.5)
SUN_HALF = 1.0


def _attn_body(xs_ref, wq_ref, wvf_ref, wvb_ref, bq_ref, bvf_ref, bvb_ref,
               wk_ref, fw_ref, fb_ref,
               rows_ref, y0_ref,
               q_s, kT_s, e_s, vf_s, vb_s, hsum_s):
    h = pl.program_id(0)
    xs = xs_ref[...]
    xs_b = xs.astype(jnp.bfloat16)

    q = jax.lax.dot_general(xs_b, wq_ref[...].astype(jnp.bfloat16),
                            (((1,), (1,)), ((), ())),
                            preferred_element_type=jnp.float32)
    q_s[...] = q + bq_ref[0]
    k = xs * wk_ref[0]
    kT_s[...] = k.T
    vf = jax.lax.dot_general(xs_b, wvf_ref[...].astype(jnp.bfloat16),
                             (((1,), (1,)), ((), ())),
                             preferred_element_type=jnp.float32) + bvf_ref[0]
    vf_s[...] = vf.astype(jnp.bfloat16)
    vb = jax.lax.dot_general(xs_b, wvb_ref[...].astype(jnp.bfloat16),
                             (((1,), (1,)), ((), ())),
                             preferred_element_type=jnp.float32) + bvb_ref[0]
    vb_s[...] = vb.astype(jnp.bfloat16)

    CH = 32

    def ib_body(ib, _):
        qb = q_s[pl.ds(ib * 8, 8), :]
        for jb in range(P // 128):
            acc = jnp.zeros((8, 128), jnp.float32)
            for wc in range(0, DM, CH):
                d = (qb[:, wc:wc + CH].reshape(8, CH, 1)
                     - kT_s[wc:wc + CH, jb * 128:(jb + 1) * 128].reshape(1, CH, 128))
                acc = acc + jnp.sum(jnp.abs(d), axis=1)
            e = jnp.exp(acc * (-SCALE))
            ii = ib * 8 + jax.lax.broadcasted_iota(jnp.int32, (8, 128), 0)
            jj = jb * 128 + jax.lax.broadcasted_iota(jnp.int32, (8, 128), 1)
            e = jnp.where((ii < NV) & (jj < NV), e, 0.0)
            e_s[pl.ds(ib * 8, 8), jb * 128:(jb + 1) * 128] = e
        return 0

    jax.lax.fori_loop(0, P // 8, ib_body, 0)

    E = e_s[...]
    denom = 1.0 + jnp.sum(E, axis=0, keepdims=True)
    A = (E / denom).astype(jnp.bfloat16)
    bf = jax.lax.dot_general(A, vf_s[...], (((0,), (0,)), ((), ())),
                             preferred_element_type=jnp.float32)
    bb = jax.lax.dot_general(A, vb_s[...], (((1,), (0,)), ((), ())),
                             preferred_element_type=jnp.float32)
    hs = bf + bb

    @pl.when(h == 0)
    def _():
        hsum_s[...] = hs

    @pl.when(h > 0)
    def _():
        hsum_s[...] = hsum_s[...] + hs

    @pl.when(h == NH - 1)
    def _():
        g = hsum_s[...] + SUN_HALF
        act = g * jax.nn.sigmoid(1.702 * g) - SUN_HALF
        y = jax.lax.dot_general(act.astype(jnp.bfloat16),
                                fw_ref[...].astype(jnp.bfloat16),
                                (((1,), (1,)), ((), ())),
                                preferred_element_type=jnp.float32) + fb_ref[...]
        rows_ref[...] = xs + y
        act0 = SUN_HALF * jax.nn.sigmoid(jnp.float32(1.702 * SUN_HALF)) - SUN_HALF
        rs = jnp.sum(fw_ref[...], axis=1)
        y0_ref[...] = (act0 * rs).reshape(1, DM) + fb_ref[...]


def _dense(xs, wqv_w, wqv_b, wk, fanin_w, fanin_b, interpret=False):
    b24 = wqv_b.reshape(3 * NH, 1, DM)
    wk3 = wk.reshape(NH, 1, DM)
    fb2 = fanin_b.reshape(1, DM)
    grid = (NH,)
    in_specs = [
        pl.BlockSpec((P, DM), lambda h: (0, 0)),
        pl.BlockSpec((DM, DM), lambda h: (h, 0)),
        pl.BlockSpec((DM, DM), lambda h: (h + NH, 0)),
        pl.BlockSpec((DM, DM), lambda h: (h + 2 * NH, 0)),
        pl.BlockSpec((1, 1, DM), lambda h: (h, 0, 0)),
        pl.BlockSpec((1, 1, DM), lambda h: (h + NH, 0, 0)),
        pl.BlockSpec((1, 1, DM), lambda h: (h + 2 * NH, 0, 0)),
        pl.BlockSpec((1, 1, DM), lambda h: (h, 0, 0)),
        pl.BlockSpec((DM, DM), lambda h: (0, 0)),
        pl.BlockSpec((1, DM), lambda h: (0, 0)),
    ]
    out_specs = [
        pl.BlockSpec((P, DM), lambda h: (0, 0)),
        pl.BlockSpec((1, DM), lambda h: (0, 0)),
    ]
    rows, y0 = pl.pallas_call(
        _attn_body,
        grid=grid,
        in_specs=in_specs,
        out_specs=out_specs,
        out_shape=[
            jax.ShapeDtypeStruct((P, DM), jnp.float32),
            jax.ShapeDtypeStruct((1, DM), jnp.float32),
        ],
        scratch_shapes=[
            pltpu.VMEM((P, DM), jnp.float32),
            pltpu.VMEM((DM, P), jnp.float32),
            pltpu.VMEM((P, P), jnp.float32),
            pltpu.VMEM((P, DM), jnp.bfloat16),
            pltpu.VMEM((P, DM), jnp.bfloat16),
            pltpu.VMEM((P, DM), jnp.float32),
        ],
        compiler_params=pltpu.CompilerParams(
            dimension_semantics=("arbitrary",),
        ),
        interpret=interpret,
    )(xs, wqv_w, wqv_w, wqv_w, b24, b24, b24, wk3, fanin_w, fb2)
    return rows, y0


def kernel(x, a2a, wk, wqv_w, wqv_b, fanin_w, fanin_b, layer, pas):
    x2 = x[0]
    idx = jnp.concatenate(
        [a2a.astype(jnp.int32), jnp.zeros((P - NV,), jnp.int32)])
    xs = x2[idx]
    rows, y0 = _dense(xs, wqv_w, wqv_b, wk, fanin_w, fanin_b)
    base = x2 + y0
    out = base.at[a2a].set(rows[:NV])
    return out[None]
